# baseline (speedup 1.0000x reference)
import jax
import jax.numpy as jnp
from jax.experimental import pallas as pl
from jax.experimental.pallas import tpu as pltpu

_TM = 400


def _fts_body(x_ref, w_ref, b_ref, out_ref):
    fts = (
        jnp.dot(x_ref[...], w_ref[...], preferred_element_type=jnp.float32)
        + b_ref[...]
    )
    out_ref[...] = fts.astype(jnp.bfloat16)


def _agg_body(fts_ref, adj_ref, out_ref):
    acc = jnp.dot(
        adj_ref[...].astype(jnp.bfloat16),
        fts_ref[...],
        preferred_element_type=jnp.float32,
    )
    out_ref[...] = jnp.maximum(acc, 0.0)


def _gcn_layer(x, adj, W, b):
    n = adj.shape[0]
    d_in, d_out = W.shape
    fts = pl.pallas_call(
        _fts_body,
        out_shape=jax.ShapeDtypeStruct((n, d_out), jnp.bfloat16),
    )(x, W, b.reshape(1, -1))
    return pl.pallas_call(
        _agg_body,
        grid=(n // _TM,),
        in_specs=[
            pl.BlockSpec((n, d_out), lambda i: (0, 0)),
            pl.BlockSpec((_TM, n), lambda i: (i, 0)),
        ],
        out_specs=pl.BlockSpec((_TM, d_out), lambda i: (i, 0)),
        out_shape=jax.ShapeDtypeStruct((n, d_out), jnp.float32),
        compiler_params=pltpu.CompilerParams(
            dimension_semantics=("parallel",),
        ),
    )(fts, adj)


def kernel(seq, adj, W1, b1, W2, b2):
    x = jnp.squeeze(seq, axis=0)
    h1 = _gcn_layer(x, adj, W1, b1)
    h2 = _gcn_layer(h1, adj, W2, b2)
    return h2[None, :, :]


# speedup vs baseline: 1.0644x; 1.0644x over previous
import jax
import jax.numpy as jnp
from jax.experimental import pallas as pl
from jax.experimental.pallas import tpu as pltpu

_TM = 400


def _body(x_ref, w1_ref, b1_ref, w2_ref, b2_ref, adj_ref, out_ref,
          fts_ref, h1_ref):
    layer = pl.program_id(0)
    i = pl.program_id(1)

    @pl.when((layer == 0) & (i == 0))
    def _():
        fts_ref[...] = (
            jnp.dot(x_ref[...].astype(jnp.bfloat16),
                    w1_ref[...].astype(jnp.bfloat16),
                    preferred_element_type=jnp.float32)
            + b1_ref[...]
        ).astype(jnp.bfloat16)

    @pl.when((layer == 1) & (i == 0))
    def _():
        fts_ref[...] = (
            jnp.dot(h1_ref[...], w2_ref[...].astype(jnp.bfloat16),
                    preferred_element_type=jnp.float32)
            + b2_ref[...]
        ).astype(jnp.bfloat16)

    act = jnp.maximum(
        jnp.dot(adj_ref[...].astype(jnp.bfloat16), fts_ref[...],
                preferred_element_type=jnp.float32),
        0.0,
    )

    @pl.when(layer == 0)
    def _():
        h1_ref[pl.ds(i * _TM, _TM), :] = act.astype(jnp.bfloat16)

    out_ref[...] = act


def kernel(seq, adj, W1, b1, W2, b2):
    x = jnp.squeeze(seq, axis=0)
    n = adj.shape[0]
    d = W1.shape[1]
    out = pl.pallas_call(
        _body,
        grid=(2, n // _TM),
        in_specs=[
            pl.BlockSpec((n, W1.shape[0]), lambda l, i: (0, 0)),
            pl.BlockSpec(W1.shape, lambda l, i: (0, 0)),
            pl.BlockSpec((1, d), lambda l, i: (0, 0)),
            pl.BlockSpec(W2.shape, lambda l, i: (0, 0)),
            pl.BlockSpec((1, d), lambda l, i: (0, 0)),
            pl.BlockSpec((_TM, n), lambda l, i: (i, 0)),
        ],
        out_specs=pl.BlockSpec((_TM, d), lambda l, i: (i, 0)),
        out_shape=jax.ShapeDtypeStruct((n, d), jnp.float32),
        scratch_shapes=[
            pltpu.VMEM((n, d), jnp.bfloat16),
            pltpu.VMEM((n, d), jnp.bfloat16),
        ],
    )(x, W1, b1.reshape(1, -1), W2, b2.reshape(1, -1), adj)
    return out[None, :, :]
